# baseline (device time: 67448 ns/iter reference)
import jax
import jax.numpy as jnp
from jax import lax
from jax.experimental import pallas as pl
from jax.experimental.pallas import tpu as pltpu

N_DEV = 4
B = 2
SQ_PER = 128
D = 512
HL = 8
DH = 64
SKV = 128
ROWS = B * SQ_PER


def kernel(x, Wq, Wo, K_ext, V_ext):
    my = lax.axis_index("i")
    K_loc = lax.dynamic_slice_in_dim(K_ext, my * HL, HL, axis=2)
    V_loc = lax.dynamic_slice_in_dim(V_ext, my * HL, HL, axis=2)
    K_loc = K_loc.transpose(0, 2, 1, 3).reshape(B * HL, SKV, DH)
    V_loc = V_loc.transpose(0, 2, 1, 3).reshape(B * HL, SKV, DH)
    x2 = x.reshape(ROWS, D)

    def body(x_ref, wq_ref, wo_ref, k_ref, v_ref, out_ref,
             xg, part, rs_recv, rs_send,
             ag_send_sems, ag_recv_sems, rs_send_sems, rs_recv_sems):
        me = lax.axis_index("i")
        left = lax.rem(me - 1 + N_DEV, N_DEV)
        right = lax.rem(me + 1, N_DEV)

        barrier_sem = pltpu.get_barrier_semaphore()
        for nbr in (left, right):
            pl.semaphore_signal(
                barrier_sem, inc=1,
                device_id=(nbr,), device_id_type=pl.DeviceIdType.MESH,
            )
        pl.semaphore_wait(barrier_sem, 2)

        xg[0] = x_ref[...]
        for h in range(N_DEV - 1):
            rdma = pltpu.make_async_remote_copy(
                src_ref=xg.at[h],
                dst_ref=xg.at[h + 1],
                send_sem=ag_send_sems.at[h],
                recv_sem=ag_recv_sems.at[h],
                device_id=(right,),
                device_id_type=pl.DeviceIdType.MESH,
            )
            rdma.start()
            rdma.wait()

        wq = wq_ref[...]
        wo = wo_ref[...]

        def compute_partial(h):
            q = jnp.dot(xg[h], wq, preferred_element_type=jnp.float32)
            brows = []
            for b in range(B):
                hcols = []
                for hl in range(HL):
                    qbh = q[b * SQ_PER:(b + 1) * SQ_PER,
                            hl * DH:(hl + 1) * DH]
                    k = k_ref[b * HL + hl]
                    v = v_ref[b * HL + hl]
                    s = lax.dot_general(
                        qbh, k, (((1,), (1,)), ((), ())),
                        preferred_element_type=jnp.float32,
                    ) * 0.125
                    m = jnp.max(s, axis=-1, keepdims=True)
                    p = jnp.exp(s - m)
                    l = jnp.sum(p, axis=-1, keepdims=True)
                    o = jnp.dot(p, v, preferred_element_type=jnp.float32) / l
                    hcols.append(o)
                brows.append(jnp.concatenate(hcols, axis=1))
            attn = jnp.concatenate(brows, axis=0)
            part[h] = jnp.dot(attn, wo, preferred_element_type=jnp.float32)

        for h in range(N_DEV):
            compute_partial(h)

        rdma0 = pltpu.make_async_remote_copy(
            src_ref=part.at[1],
            dst_ref=rs_recv.at[0],
            send_sem=rs_send_sems.at[0],
            recv_sem=rs_recv_sems.at[0],
            device_id=(right,),
            device_id_type=pl.DeviceIdType.MESH,
        )
        rdma0.start()
        rdma0.wait()
        for t in (1, 2):
            rs_send[t - 1] = rs_recv[t - 1] + part[t + 1]
            rdma = pltpu.make_async_remote_copy(
                src_ref=rs_send.at[t - 1],
                dst_ref=rs_recv.at[t],
                send_sem=rs_send_sems.at[t],
                recv_sem=rs_recv_sems.at[t],
                device_id=(right,),
                device_id_type=pl.DeviceIdType.MESH,
            )
            rdma.start()
            rdma.wait()

        out_ref[...] = rs_recv[2] + part[0]

    out2 = pl.pallas_call(
        body,
        out_shape=jax.ShapeDtypeStruct((ROWS, D), jnp.float32),
        in_specs=[pl.BlockSpec(memory_space=pltpu.VMEM)] * 5,
        out_specs=pl.BlockSpec(memory_space=pltpu.VMEM),
        scratch_shapes=[
            pltpu.VMEM((N_DEV, ROWS, D), jnp.float32),
            pltpu.VMEM((N_DEV, ROWS, D), jnp.float32),
            pltpu.VMEM((3, ROWS, D), jnp.float32),
            pltpu.VMEM((2, ROWS, D), jnp.float32),
            pltpu.SemaphoreType.DMA((3,)),
            pltpu.SemaphoreType.DMA((3,)),
            pltpu.SemaphoreType.DMA((3,)),
            pltpu.SemaphoreType.DMA((3,)),
        ],
        compiler_params=pltpu.CompilerParams(collective_id=0),
    )(x2, Wq, Wo, K_loc, V_loc)
    return out2.reshape(B, SQ_PER, D)


# device time: 49465 ns/iter; 1.3635x vs baseline; 1.3635x over previous
import jax
import jax.numpy as jnp
from jax import lax
from jax.experimental import pallas as pl
from jax.experimental.pallas import tpu as pltpu

N_DEV = 4
B = 2
SQ_PER = 128
D = 512
HL = 8
DH = 64
SKV = 128
ROWS = B * SQ_PER


def kernel(x, Wq, Wo, K_ext, V_ext):
    my = lax.axis_index("i")
    K_loc = lax.dynamic_slice_in_dim(K_ext, my * HL, HL, axis=2)
    V_loc = lax.dynamic_slice_in_dim(V_ext, my * HL, HL, axis=2)
    K_loc = K_loc.transpose(0, 2, 1, 3).reshape(B * HL, SKV, DH)
    V_loc = V_loc.transpose(0, 2, 1, 3).reshape(B * HL, SKV, DH)
    x2 = x.reshape(ROWS, D)

    def body(x_ref, wq_ref, wo_ref, k_ref, v_ref, out_ref,
             xg, part, rs_recv, rs_send,
             ag_send_sems, ag_recv_sems, rs_send_sems, rs_recv_sems):
        me = lax.axis_index("i")
        left = lax.rem(me - 1 + N_DEV, N_DEV)
        right = lax.rem(me + 1, N_DEV)

        barrier_sem = pltpu.get_barrier_semaphore()
        for nbr in (left, right):
            pl.semaphore_signal(
                barrier_sem, inc=1,
                device_id=(nbr,), device_id_type=pl.DeviceIdType.MESH,
            )
        pl.semaphore_wait(barrier_sem, 2)

        def copy(src, dst, ssem, rsem):
            return pltpu.make_async_remote_copy(
                src_ref=src, dst_ref=dst, send_sem=ssem, recv_sem=rsem,
                device_id=(right,), device_id_type=pl.DeviceIdType.MESH,
            )

        wq = wq_ref[...]
        wo = wo_ref[...]

        def compute_partial(h):
            q = jnp.dot(xg[h], wq, preferred_element_type=jnp.float32)
            brows = []
            for b in range(B):
                hcols = []
                for hl in range(HL):
                    qbh = q[b * SQ_PER:(b + 1) * SQ_PER,
                            hl * DH:(hl + 1) * DH]
                    k = k_ref[b * HL + hl]
                    v = v_ref[b * HL + hl]
                    s = lax.dot_general(
                        qbh, k, (((1,), (1,)), ((), ())),
                        preferred_element_type=jnp.float32,
                    ) * 0.125
                    m = jnp.max(s, axis=-1, keepdims=True)
                    p = jnp.exp(s - m)
                    l = jnp.sum(p, axis=-1, keepdims=True)
                    o = jnp.dot(p, v, preferred_element_type=jnp.float32) / l
                    hcols.append(o)
                brows.append(jnp.concatenate(hcols, axis=1))
            attn = jnp.concatenate(brows, axis=0)
            part[h] = jnp.dot(attn, wo, preferred_element_type=jnp.float32)

        xg[0] = x_ref[...]
        drains = []

        ag0 = copy(xg.at[0], xg.at[1], ag_send_sems.at[0], ag_recv_sems.at[0])
        ag0.start()
        drains.append(ag0)
        compute_partial(0)
        ag0.wait_recv()

        ag1 = copy(xg.at[1], xg.at[2], ag_send_sems.at[1], ag_recv_sems.at[1])
        ag1.start()
        drains.append(ag1)
        compute_partial(1)
        ag1.wait_recv()

        ag2 = copy(xg.at[2], xg.at[3], ag_send_sems.at[2], ag_recv_sems.at[2])
        ag2.start()
        drains.append(ag2)
        rs0 = copy(part.at[1], rs_recv.at[0],
                   rs_send_sems.at[0], rs_recv_sems.at[0])
        rs0.start()
        drains.append(rs0)
        compute_partial(2)
        ag2.wait_recv()
        compute_partial(3)

        rs0.wait_recv()
        rs_send[0] = rs_recv[0] + part[2]
        rs1 = copy(rs_send.at[0], rs_recv.at[1],
                   rs_send_sems.at[1], rs_recv_sems.at[1])
        rs1.start()
        drains.append(rs1)
        rs1.wait_recv()
        rs_send[1] = rs_recv[1] + part[3]
        rs2 = copy(rs_send.at[1], rs_recv.at[2],
                   rs_send_sems.at[2], rs_recv_sems.at[2])
        rs2.start()
        drains.append(rs2)
        rs2.wait_recv()

        out_ref[...] = rs_recv[2] + part[0]
        for d in drains:
            d.wait_send()

    out2 = pl.pallas_call(
        body,
        out_shape=jax.ShapeDtypeStruct((ROWS, D), jnp.float32),
        in_specs=[pl.BlockSpec(memory_space=pltpu.VMEM)] * 5,
        out_specs=pl.BlockSpec(memory_space=pltpu.VMEM),
        scratch_shapes=[
            pltpu.VMEM((N_DEV, ROWS, D), jnp.float32),
            pltpu.VMEM((N_DEV, ROWS, D), jnp.float32),
            pltpu.VMEM((3, ROWS, D), jnp.float32),
            pltpu.VMEM((2, ROWS, D), jnp.float32),
            pltpu.SemaphoreType.DMA((3,)),
            pltpu.SemaphoreType.DMA((3,)),
            pltpu.SemaphoreType.DMA((3,)),
            pltpu.SemaphoreType.DMA((3,)),
        ],
        compiler_params=pltpu.CompilerParams(collective_id=0),
    )(x2, Wq, Wo, K_loc, V_loc)
    return out2.reshape(B, SQ_PER, D)


# device time: 34601 ns/iter; 1.9493x vs baseline; 1.4296x over previous
import jax
import jax.numpy as jnp
from jax import lax
from jax.experimental import pallas as pl
from jax.experimental.pallas import tpu as pltpu

N_DEV = 4
B = 2
SQ_PER = 128
D = 512
HL = 8
DH = 64
SKV = 128
ROWS = B * SQ_PER


def kernel(x, Wq, Wo, K_ext, V_ext):
    my = lax.axis_index("i")
    K_loc = lax.dynamic_slice_in_dim(K_ext, my * HL, HL, axis=2)
    V_loc = lax.dynamic_slice_in_dim(V_ext, my * HL, HL, axis=2)
    K_loc = K_loc.transpose(0, 2, 1, 3).reshape(B * HL, SKV, DH)
    V_loc = V_loc.transpose(0, 2, 1, 3).reshape(B * HL, SKV, DH)
    x2 = x.reshape(ROWS, D)

    def body(x_ref, wq_ref, wo_ref, k_ref, v_ref, out_ref,
             xgR, xgL, partR, partL, rsR_recv, rsL_recv, rsR_send, rsL_send,
             agR_ssem, agR_rsem, agL_ssem, agL_rsem,
             rsR_ssem, rsR_rsem, rsL_ssem, rsL_rsem):
        me = lax.axis_index("i")
        left = lax.rem(me - 1 + N_DEV, N_DEV)
        right = lax.rem(me + 1, N_DEV)

        barrier_sem = pltpu.get_barrier_semaphore()
        for nbr in (left, right):
            pl.semaphore_signal(
                barrier_sem, inc=1,
                device_id=(nbr,), device_id_type=pl.DeviceIdType.MESH,
            )
        pl.semaphore_wait(barrier_sem, 2)

        def copy(src, dst, ssem, rsem, to):
            return pltpu.make_async_remote_copy(
                src_ref=src, dst_ref=dst, send_sem=ssem, recv_sem=rsem,
                device_id=(to,), device_id_type=pl.DeviceIdType.MESH,
            )

        wq = wq_ref[...]
        wo = wo_ref[...]

        def compute_partial(xg_s, part_s, h, b):
            q = jnp.dot(xg_s[h], wq, preferred_element_type=jnp.float32)
            hcols = []
            for hl in range(HL):
                qbh = q[:, hl * DH:(hl + 1) * DH]
                k = k_ref[b * HL + hl]
                v = v_ref[b * HL + hl]
                s = lax.dot_general(
                    qbh, k, (((1,), (1,)), ((), ())),
                    preferred_element_type=jnp.float32,
                ) * 0.125
                m = jnp.max(s, axis=-1, keepdims=True)
                p = jnp.exp(s - m)
                l = jnp.sum(p, axis=-1, keepdims=True)
                o = jnp.dot(p, v, preferred_element_type=jnp.float32) / l
                hcols.append(o)
            attn = jnp.concatenate(hcols, axis=1)
            part_s[h] = jnp.dot(attn, wo, preferred_element_type=jnp.float32)

        def compute_both(h):
            compute_partial(xgR, partR, h, 0)
            compute_partial(xgL, partL, h, 1)

        xgR[0] = x_ref[0:SQ_PER, :]
        xgL[0] = x_ref[SQ_PER:ROWS, :]
        drains = []

        def start(rdma):
            rdma.start()
            drains.append(rdma)
            return rdma

        def ag_hop(h):
            r = start(copy(xgR.at[h], xgR.at[h + 1],
                           agR_ssem.at[h], agR_rsem.at[h], right))
            l = start(copy(xgL.at[h], xgL.at[h + 1],
                           agL_ssem.at[h], agL_rsem.at[h], left))
            return r, l

        agR0, agL0 = ag_hop(0)
        compute_both(0)
        agR0.wait_recv()
        agL0.wait_recv()

        agR1, agL1 = ag_hop(1)
        compute_both(1)
        agR1.wait_recv()
        agL1.wait_recv()

        agR2, agL2 = ag_hop(2)
        rsR0 = start(copy(partR.at[1], rsR_recv.at[0],
                          rsR_ssem.at[0], rsR_rsem.at[0], right))
        rsL0 = start(copy(partL.at[1], rsL_recv.at[0],
                          rsL_ssem.at[0], rsL_rsem.at[0], left))
        compute_both(2)
        agR2.wait_recv()
        agL2.wait_recv()
        compute_both(3)

        rsR0.wait_recv()
        rsL0.wait_recv()
        rsR_send[0] = rsR_recv[0] + partR[2]
        rsL_send[0] = rsL_recv[0] + partL[2]
        rsR1 = start(copy(rsR_send.at[0], rsR_recv.at[1],
                          rsR_ssem.at[1], rsR_rsem.at[1], right))
        rsL1 = start(copy(rsL_send.at[0], rsL_recv.at[1],
                          rsL_ssem.at[1], rsL_rsem.at[1], left))
        rsR1.wait_recv()
        rsL1.wait_recv()
        rsR_send[1] = rsR_recv[1] + partR[3]
        rsL_send[1] = rsL_recv[1] + partL[3]
        rsR2 = start(copy(rsR_send.at[1], rsR_recv.at[2],
                          rsR_ssem.at[2], rsR_rsem.at[2], right))
        rsL2 = start(copy(rsL_send.at[1], rsL_recv.at[2],
                          rsL_ssem.at[2], rsL_rsem.at[2], left))
        rsR2.wait_recv()
        rsL2.wait_recv()

        out_ref[0:SQ_PER, :] = rsR_recv[2] + partR[0]
        out_ref[SQ_PER:ROWS, :] = rsL_recv[2] + partL[0]
        for d in drains:
            d.wait_send()

    half = (SQ_PER, D)
    out2 = pl.pallas_call(
        body,
        out_shape=jax.ShapeDtypeStruct((ROWS, D), jnp.float32),
        in_specs=[pl.BlockSpec(memory_space=pltpu.VMEM)] * 5,
        out_specs=pl.BlockSpec(memory_space=pltpu.VMEM),
        scratch_shapes=[
            pltpu.VMEM((N_DEV,) + half, jnp.float32),
            pltpu.VMEM((N_DEV,) + half, jnp.float32),
            pltpu.VMEM((N_DEV,) + half, jnp.float32),
            pltpu.VMEM((N_DEV,) + half, jnp.float32),
            pltpu.VMEM((3,) + half, jnp.float32),
            pltpu.VMEM((3,) + half, jnp.float32),
            pltpu.VMEM((2,) + half, jnp.float32),
            pltpu.VMEM((2,) + half, jnp.float32),
        ] + [pltpu.SemaphoreType.DMA((3,))] * 8,
        compiler_params=pltpu.CompilerParams(collective_id=0),
    )(x2, Wq, Wo, K_loc, V_loc)
    return out2.reshape(B, SQ_PER, D)


# device time: 32957 ns/iter; 2.0465x vs baseline; 1.0499x over previous
import jax
import jax.numpy as jnp
from jax import lax
from jax.experimental import pallas as pl
from jax.experimental.pallas import tpu as pltpu

N_DEV = 4
B = 2
SQ_PER = 128
D = 512
HL = 8
DH = 64
SKV = 128
ROWS = B * SQ_PER


def kernel(x, Wq, Wo, K_ext, V_ext):
    my = lax.axis_index("i")
    K_loc = lax.dynamic_slice_in_dim(K_ext, my * HL, HL, axis=2)
    V_loc = lax.dynamic_slice_in_dim(V_ext, my * HL, HL, axis=2)
    K_loc = K_loc.transpose(0, 2, 1, 3).reshape(B * HL, SKV, DH)
    V_loc = V_loc.transpose(0, 2, 1, 3).reshape(B * HL, SKV, DH)
    x2 = x.reshape(ROWS, D)

    def body(x_ref, wq_ref, wo_ref, k_ref, v_ref, out_ref,
             xgR, xgL, partR, partL, rsR_recv, rsL_recv, rsR_send, rsL_send,
             agR_ssem, agR_rsem, agL_ssem, agL_rsem,
             rsR_ssem, rsR_rsem, rsL_ssem, rsL_rsem):
        me = lax.axis_index("i")
        left = lax.rem(me - 1 + N_DEV, N_DEV)
        right = lax.rem(me + 1, N_DEV)

        barrier_sem = pltpu.get_barrier_semaphore()
        for nbr in (left, right):
            pl.semaphore_signal(
                barrier_sem, inc=1,
                device_id=(nbr,), device_id_type=pl.DeviceIdType.MESH,
            )
        pl.semaphore_wait(barrier_sem, 2)

        def copy(src, dst, ssem, rsem, to):
            return pltpu.make_async_remote_copy(
                src_ref=src, dst_ref=dst, send_sem=ssem, recv_sem=rsem,
                device_id=(to,), device_id_type=pl.DeviceIdType.MESH,
            )

        wq = wq_ref[...]
        wo = wo_ref[...]

        def compute_partial(xg_s, part_s, h, b):
            q = jnp.dot(xg_s[h], wq, preferred_element_type=jnp.float32)
            hcols = []
            for hl in range(HL):
                qbh = q[:, hl * DH:(hl + 1) * DH]
                k = k_ref[b * HL + hl]
                v = v_ref[b * HL + hl]
                s = lax.dot_general(
                    qbh, k, (((1,), (1,)), ((), ())),
                    preferred_element_type=jnp.float32,
                ) * 0.125
                m = jnp.max(s, axis=-1, keepdims=True)
                p = jnp.exp(s - m)
                l = jnp.sum(p, axis=-1, keepdims=True)
                o = jnp.dot(p, v, preferred_element_type=jnp.float32) / l
                hcols.append(o)
            attn = jnp.concatenate(hcols, axis=1)
            part_s[h] = jnp.dot(attn, wo, preferred_element_type=jnp.float32)

        def compute_both(h):
            compute_partial(xgR, partR, h, 0)
            compute_partial(xgL, partL, h, 1)

        xgR[0] = x_ref[0:SQ_PER, :]
        xgL[0] = x_ref[SQ_PER:ROWS, :]
        drains = []

        def start(rdma):
            rdma.start()
            drains.append(rdma)
            return rdma

        def ag_hop(h):
            r = start(copy(xgR.at[h], xgR.at[h + 1],
                           agR_ssem.at[h], agR_rsem.at[h], right))
            l = start(copy(xgL.at[h], xgL.at[h + 1],
                           agL_ssem.at[h], agL_rsem.at[h], left))
            return r, l

        agR0, agL0 = ag_hop(0)
        compute_both(0)
        agR0.wait_recv()
        agL0.wait_recv()

        agR1, agL1 = ag_hop(1)
        compute_both(1)
        agR1.wait_recv()
        agL1.wait_recv()

        agR2, agL2 = ag_hop(2)
        rsR0 = start(copy(partR.at[1], rsR_recv.at[0],
                          rsR_ssem.at[0], rsR_rsem.at[0], right))
        rsL0 = start(copy(partL.at[1], rsL_recv.at[0],
                          rsL_ssem.at[0], rsL_rsem.at[0], left))
        compute_both(2)
        agR2.wait_recv()
        agL2.wait_recv()

        rsR0.wait_recv()
        rsL0.wait_recv()
        rsR_send[0] = rsR_recv[0] + partR[2]
        rsL_send[0] = rsL_recv[0] + partL[2]
        rsR1 = start(copy(rsR_send.at[0], rsR_recv.at[1],
                          rsR_ssem.at[1], rsR_rsem.at[1], right))
        rsL1 = start(copy(rsL_send.at[0], rsL_recv.at[1],
                          rsL_ssem.at[1], rsL_rsem.at[1], left))
        compute_both(3)
        rsR1.wait_recv()
        rsL1.wait_recv()
        rsR_send[1] = rsR_recv[1] + partR[3]
        rsL_send[1] = rsL_recv[1] + partL[3]
        rsR2 = start(copy(rsR_send.at[1], rsR_recv.at[2],
                          rsR_ssem.at[2], rsR_rsem.at[2], right))
        rsL2 = start(copy(rsL_send.at[1], rsL_recv.at[2],
                          rsL_ssem.at[2], rsL_rsem.at[2], left))
        rsR2.wait_recv()
        rsL2.wait_recv()

        out_ref[0:SQ_PER, :] = rsR_recv[2] + partR[0]
        out_ref[SQ_PER:ROWS, :] = rsL_recv[2] + partL[0]
        for d in drains:
            d.wait_send()

    half = (SQ_PER, D)
    out2 = pl.pallas_call(
        body,
        out_shape=jax.ShapeDtypeStruct((ROWS, D), jnp.float32),
        in_specs=[pl.BlockSpec(memory_space=pltpu.VMEM)] * 5,
        out_specs=pl.BlockSpec(memory_space=pltpu.VMEM),
        scratch_shapes=[
            pltpu.VMEM((N_DEV,) + half, jnp.float32),
            pltpu.VMEM((N_DEV,) + half, jnp.float32),
            pltpu.VMEM((N_DEV,) + half, jnp.float32),
            pltpu.VMEM((N_DEV,) + half, jnp.float32),
            pltpu.VMEM((3,) + half, jnp.float32),
            pltpu.VMEM((3,) + half, jnp.float32),
            pltpu.VMEM((2,) + half, jnp.float32),
            pltpu.VMEM((2,) + half, jnp.float32),
        ] + [pltpu.SemaphoreType.DMA((3,))] * 8,
        compiler_params=pltpu.CompilerParams(collective_id=0),
    )(x2, Wq, Wo, K_loc, V_loc)
    return out2.reshape(B, SQ_PER, D)


# device time: 30525 ns/iter; 2.2096x vs baseline; 1.0797x over previous
import jax
import jax.numpy as jnp
from jax import lax
from jax.experimental import pallas as pl
from jax.experimental.pallas import tpu as pltpu

N_DEV = 4
B = 2
SQ_PER = 128
D = 512
HL = 8
DH = 64
SKV = 128
ROWS = B * SQ_PER


def kernel(x, Wq, Wo, K_ext, V_ext):
    my = lax.axis_index("i")
    K_loc = lax.dynamic_slice_in_dim(K_ext, my * HL, HL, axis=2)
    V_loc = lax.dynamic_slice_in_dim(V_ext, my * HL, HL, axis=2)
    K_loc = K_loc.transpose(0, 2, 1, 3).reshape(B * HL, SKV, DH)
    V_loc = V_loc.transpose(0, 2, 1, 3).reshape(B * HL, SKV, DH)
    x2 = x.reshape(ROWS, D)

    def body(x_ref, wq_ref, wo_ref, k_ref, v_ref, out_ref,
             xgR, xgL, partR, partL, rsR_recv, rsL_recv, rsR_send, rsL_send,
             agR_ssem, agR_rsem, agL_ssem, agL_rsem,
             rsR_ssem, rsR_rsem, rsL_ssem, rsL_rsem):
        me = lax.axis_index("i")
        left = lax.rem(me - 1 + N_DEV, N_DEV)
        right = lax.rem(me + 1, N_DEV)

        barrier_sem = pltpu.get_barrier_semaphore()
        for nbr in (left, right):
            pl.semaphore_signal(
                barrier_sem, inc=1,
                device_id=(nbr,), device_id_type=pl.DeviceIdType.MESH,
            )
        pl.semaphore_wait(barrier_sem, 2)

        def copy(src, dst, ssem, rsem, to):
            return pltpu.make_async_remote_copy(
                src_ref=src, dst_ref=dst, send_sem=ssem, recv_sem=rsem,
                device_id=(to,), device_id_type=pl.DeviceIdType.MESH,
            )

        bf16 = jnp.bfloat16
        wq = wq_ref[...].astype(bf16)
        wo = wo_ref[...].astype(bf16)

        def compute_partial(xg_s, part_s, h, b):
            q = jnp.dot(xg_s[h], wq,
                        preferred_element_type=jnp.float32).astype(bf16)
            hcols = []
            for hl in range(HL):
                qbh = q[:, hl * DH:(hl + 1) * DH]
                k = k_ref[b * HL + hl].astype(bf16)
                v = v_ref[b * HL + hl].astype(bf16)
                s = lax.dot_general(
                    qbh, k, (((1,), (1,)), ((), ())),
                    preferred_element_type=jnp.float32,
                ) * 0.125
                m = jnp.max(s, axis=-1, keepdims=True)
                p = jnp.exp(s - m)
                l = jnp.sum(p, axis=-1, keepdims=True)
                o = jnp.dot(p.astype(bf16), v,
                            preferred_element_type=jnp.float32) / l
                hcols.append(o.astype(bf16))
            attn = jnp.concatenate(hcols, axis=1)
            part_s[h] = jnp.dot(
                attn, wo, preferred_element_type=jnp.float32).astype(bf16)

        def compute_both(h):
            compute_partial(xgR, partR, h, 0)
            compute_partial(xgL, partL, h, 1)

        xgR[0] = x_ref[0:SQ_PER, :].astype(bf16)
        xgL[0] = x_ref[SQ_PER:ROWS, :].astype(bf16)
        drains = []

        def start(rdma):
            rdma.start()
            drains.append(rdma)
            return rdma

        def ag_hop(h):
            r = start(copy(xgR.at[h], xgR.at[h + 1],
                           agR_ssem.at[h], agR_rsem.at[h], right))
            l = start(copy(xgL.at[h], xgL.at[h + 1],
                           agL_ssem.at[h], agL_rsem.at[h], left))
            return r, l

        agR0, agL0 = ag_hop(0)
        compute_both(0)
        agR0.wait_recv()
        agL0.wait_recv()

        agR1, agL1 = ag_hop(1)
        compute_both(1)
        agR1.wait_recv()
        agL1.wait_recv()

        agR2, agL2 = ag_hop(2)
        rsR0 = start(copy(partR.at[1], rsR_recv.at[0],
                          rsR_ssem.at[0], rsR_rsem.at[0], right))
        rsL0 = start(copy(partL.at[1], rsL_recv.at[0],
                          rsL_ssem.at[0], rsL_rsem.at[0], left))
        compute_both(2)
        agR2.wait_recv()
        agL2.wait_recv()

        rsR0.wait_recv()
        rsL0.wait_recv()
        rsR_send[0] = rsR_recv[0] + partR[2]
        rsL_send[0] = rsL_recv[0] + partL[2]
        rsR1 = start(copy(rsR_send.at[0], rsR_recv.at[1],
                          rsR_ssem.at[1], rsR_rsem.at[1], right))
        rsL1 = start(copy(rsL_send.at[0], rsL_recv.at[1],
                          rsL_ssem.at[1], rsL_rsem.at[1], left))
        compute_both(3)
        rsR1.wait_recv()
        rsL1.wait_recv()
        rsR_send[1] = rsR_recv[1] + partR[3]
        rsL_send[1] = rsL_recv[1] + partL[3]
        rsR2 = start(copy(rsR_send.at[1], rsR_recv.at[2],
                          rsR_ssem.at[2], rsR_rsem.at[2], right))
        rsL2 = start(copy(rsL_send.at[1], rsL_recv.at[2],
                          rsL_ssem.at[2], rsL_rsem.at[2], left))
        rsR2.wait_recv()
        rsL2.wait_recv()

        out_ref[0:SQ_PER, :] = (rsR_recv[2].astype(jnp.float32)
                                + partR[0].astype(jnp.float32))
        out_ref[SQ_PER:ROWS, :] = (rsL_recv[2].astype(jnp.float32)
                                   + partL[0].astype(jnp.float32))
        for d in drains:
            d.wait_send()

    half = (SQ_PER, D)
    out2 = pl.pallas_call(
        body,
        out_shape=jax.ShapeDtypeStruct((ROWS, D), jnp.float32),
        in_specs=[pl.BlockSpec(memory_space=pltpu.VMEM)] * 5,
        out_specs=pl.BlockSpec(memory_space=pltpu.VMEM),
        scratch_shapes=[
            pltpu.VMEM((N_DEV,) + half, jnp.bfloat16),
            pltpu.VMEM((N_DEV,) + half, jnp.bfloat16),
            pltpu.VMEM((N_DEV,) + half, jnp.bfloat16),
            pltpu.VMEM((N_DEV,) + half, jnp.bfloat16),
            pltpu.VMEM((3,) + half, jnp.bfloat16),
            pltpu.VMEM((3,) + half, jnp.bfloat16),
            pltpu.VMEM((2,) + half, jnp.bfloat16),
            pltpu.VMEM((2,) + half, jnp.bfloat16),
        ] + [pltpu.SemaphoreType.DMA((3,))] * 8,
        compiler_params=pltpu.CompilerParams(collective_id=0),
    )(x2, Wq, Wo, K_loc, V_loc)
    return out2.reshape(B, SQ_PER, D)


# device time: 21166 ns/iter; 3.1866x vs baseline; 1.4422x over previous
import jax
import jax.numpy as jnp
from jax import lax
from jax.experimental import pallas as pl
from jax.experimental.pallas import tpu as pltpu

N_DEV = 4
B = 2
SQ_PER = 128
D = 512
HL = 8
DH = 64
SKV = 128
ROWS = B * SQ_PER


def kernel(x, Wq, Wo, K_ext, V_ext):
    my = lax.axis_index("i")
    K_loc = lax.dynamic_slice_in_dim(K_ext, my * HL, HL, axis=2)
    V_loc = lax.dynamic_slice_in_dim(V_ext, my * HL, HL, axis=2)
    K_loc = K_loc.transpose(0, 2, 1, 3).reshape(B * HL, SKV, DH)
    V_loc = V_loc.transpose(0, 2, 1, 3).reshape(B * HL, SKV, DH)
    x2 = x.reshape(ROWS, D)

    def body(x_ref, wq_ref, wo_ref, k_ref, v_ref, out_ref,
             xbuf, pbuf, rbuf, ag_ssem, ag_rsem, rs_ssem, rs_rsem):
        me = lax.axis_index("i")
        left = lax.rem(me - 1 + N_DEV, N_DEV)
        right = lax.rem(me + 1, N_DEV)
        diag = lax.rem(me + 2, N_DEV)

        xbuf[0] = x_ref[...].astype(jnp.bfloat16)

        barrier_sem = pltpu.get_barrier_semaphore()
        for nbr in (left, right):
            pl.semaphore_signal(
                barrier_sem, inc=1,
                device_id=(nbr,), device_id_type=pl.DeviceIdType.MESH,
            )
        pl.semaphore_wait(barrier_sem, 2)

        def copy(src, dst, ssem, rsem, to):
            return pltpu.make_async_remote_copy(
                src_ref=src, dst_ref=dst, send_sem=ssem, recv_sem=rsem,
                device_id=(to,), device_id_type=pl.DeviceIdType.MESH,
            )

        bf16 = jnp.bfloat16
        wq = wq_ref[...].astype(bf16)
        wo = wo_ref[...].astype(bf16)
        wo3 = wo.reshape(HL, DH, D)
        kb = [(k_ref[b * HL:(b + 1) * HL] * 0.125).astype(bf16)
              for b in range(B)]
        vb = [v_ref[b * HL:(b + 1) * HL].astype(bf16) for b in range(B)]

        def rows(b):
            return slice(b * SQ_PER, (b + 1) * SQ_PER)

        def compute_half(slot, b):
            q = jnp.dot(xbuf[slot, rows(b)], wq,
                        preferred_element_type=jnp.float32).astype(bf16)
            q3 = q.reshape(SQ_PER, HL, DH)
            s = lax.dot_general(
                q3, kb[b], (((2,), (2,)), ((1,), (0,))),
                preferred_element_type=jnp.float32)
            p = jnp.exp(s)
            linv = 1.0 / jnp.sum(p, axis=-1, keepdims=True)
            o = lax.dot_general(
                p.astype(bf16), vb[b], (((2,), (1,)), ((0,), (0,))),
                preferred_element_type=jnp.float32) * linv
            po = lax.dot_general(
                o.astype(bf16), wo3, (((2,), (1,)), ((0,), (0,))),
                preferred_element_type=jnp.float32)
            return jnp.sum(po, axis=0).astype(bf16)

        drains = []

        def start(rdma):
            rdma.start()
            drains.append(rdma)
            return rdma

        def ag_copy(dst_slot, b, to):
            return copy(xbuf.at[0, rows(b)], xbuf.at[dst_slot, rows(b)],
                        ag_ssem.at[dst_slot - 1, b],
                        ag_rsem.at[dst_slot, b], to)

        ag_r0 = start(ag_copy(1, 0, right))
        ag_l0 = start(ag_copy(2, 0, left))
        start(ag_copy(1, 1, right))
        ag_l1 = start(ag_copy(2, 1, left))

        own0 = compute_half(0, 0)

        def rs_send(i, b, peer_slot, to):
            start(copy(pbuf.at[i, b], rbuf.at[peer_slot, b],
                       rs_ssem.at[i, b], rs_rsem.at[peer_slot, b], to))

        ag_r0.wait_recv()
        ag_d0 = start(ag_copy(3, 0, diag))
        start(ag_copy(3, 1, diag))

        pbuf[0, 0] = compute_half(1, 0)
        rs_send(0, 0, 1, left)
        ag_copy(1, 1, left).wait_recv()
        pbuf[0, 1] = compute_half(1, 1)
        rs_send(0, 1, 1, left)

        ag_l0.wait_recv()
        pbuf[1, 0] = compute_half(2, 0)
        rs_send(1, 0, 0, right)

        ag_d0.wait_recv()
        pbuf[2, 0] = compute_half(3, 0)
        rs_send(2, 0, 2, diag)
        ag_copy(3, 1, diag).wait_recv()
        pbuf[2, 1] = compute_half(3, 1)
        rs_send(2, 1, 2, diag)

        ag_l1.wait_recv()
        pbuf[1, 1] = compute_half(2, 1)
        rs_send(1, 1, 0, right)

        own1 = compute_half(0, 1)

        def rwait(i, b):
            copy(pbuf.at[i, b], rbuf.at[i, b],
                 rs_ssem.at[i, b], rs_rsem.at[i, b], left).wait_recv()

        acc = [own0.astype(jnp.float32), own1.astype(jnp.float32)]
        for i in range(2):
            for b in range(B):
                rwait(i, b)
                acc[b] = acc[b] + rbuf[i, b].astype(jnp.float32)
        for b in range(B):
            rwait(2, b)
            out_ref[rows(b), :] = acc[b] + rbuf[2, b].astype(jnp.float32)
        for d in drains:
            d.wait_send()

    halfsh = (SQ_PER, D)
    out2 = pl.pallas_call(
        body,
        out_shape=jax.ShapeDtypeStruct((ROWS, D), jnp.float32),
        in_specs=[pl.BlockSpec(memory_space=pltpu.VMEM)] * 5,
        out_specs=pl.BlockSpec(memory_space=pltpu.VMEM),
        scratch_shapes=[
            pltpu.VMEM((N_DEV, ROWS, D), jnp.bfloat16),
            pltpu.VMEM((3, B) + halfsh, jnp.bfloat16),
            pltpu.VMEM((3, B) + halfsh, jnp.bfloat16),
            pltpu.SemaphoreType.DMA((3, B)),
            pltpu.SemaphoreType.DMA((N_DEV, B)),
            pltpu.SemaphoreType.DMA((3, B)),
            pltpu.SemaphoreType.DMA((3, B)),
        ],
        compiler_params=pltpu.CompilerParams(collective_id=0),
    )(x2, Wq, Wo, K_loc, V_loc)
    return out2.reshape(B, SQ_PER, D)
